# Initial kernel scaffold; baseline (speedup 1.0000x reference)
import jax
import jax.numpy as jnp
from jax.experimental import pallas as pl


def kernel(x, table):
    raise NotImplementedError("write your pallas kernel here")


# speedup vs baseline: 1.4781x; 1.4781x over previous
import functools

import jax
import jax.numpy as jnp
from jax import lax
from jax.experimental import pallas as pl
from jax.experimental.pallas import tpu as pltpu
from jax.experimental.pallas import tpu_sc as plsc


def _gather_kernel(n_rows, d_model, chunk, n_chunks, nc):
    mesh = plsc.VectorSubcoreMesh(core_axis_name="c", subcore_axis_name="s")

    @functools.partial(
        pl.kernel,
        mesh=mesh,
        out_type=jax.ShapeDtypeStruct((n_rows, d_model), jnp.float32),
        compiler_params=pltpu.CompilerParams(use_tc_tiling_on_sc=False),
        scratch_types=[
            pltpu.VMEM((chunk,), jnp.int32),
            pltpu.VMEM((chunk, d_model), jnp.float32),
            pltpu.SemaphoreType.DMA,
        ],
    )
    def k(idx_hbm, table_hbm, out_hbm, idx_v, rows_v, sem):
        wid = lax.axis_index("s") * nc + lax.axis_index("c")
        base = wid * (chunk * n_chunks)

        def body(i, carry):
            start = base + i * chunk
            pltpu.sync_copy(idx_hbm.at[pl.ds(start, chunk)], idx_v)
            pltpu.async_copy(table_hbm.at[idx_v], rows_v, sem).wait()
            pltpu.sync_copy(rows_v, out_hbm.at[pl.ds(start, chunk)])
            return carry

        lax.fori_loop(0, n_chunks, body, 0)

    return k


def kernel(x, table):
    batch, hist = x.shape
    vocab, d_model = table.shape
    n_rows = batch * hist

    info = plsc.get_sparse_core_info()
    nw = info.num_cores * info.num_subcores
    rows_per_w = n_rows // nw
    chunk = 1600
    n_chunks = rows_per_w // chunk

    idx = x.reshape(n_rows).astype(jnp.int32)
    k = _gather_kernel(n_rows, d_model, chunk, n_chunks, info.num_cores)
    out = k(idx, table)
    return out.reshape(batch, hist, d_model)


# speedup vs baseline: 1.4922x; 1.0096x over previous
import functools

import jax
import jax.numpy as jnp
from jax import lax
from jax.experimental import pallas as pl
from jax.experimental.pallas import tpu as pltpu
from jax.experimental.pallas import tpu_sc as plsc

_NBUF = 2


def _gather_kernel(n_rows, d_model, chunk, n_chunks, nc):
    mesh = plsc.VectorSubcoreMesh(core_axis_name="c", subcore_axis_name="s")
    n_super = n_chunks // _NBUF

    @functools.partial(
        pl.kernel,
        mesh=mesh,
        out_type=jax.ShapeDtypeStruct((n_rows, d_model), jnp.float32),
        compiler_params=pltpu.CompilerParams(use_tc_tiling_on_sc=False),
        scratch_types=[
            pltpu.VMEM((_NBUF, chunk), jnp.int32),
            pltpu.VMEM((_NBUF, chunk, d_model), jnp.float32),
            [pltpu.SemaphoreType.DMA] * _NBUF,
            [pltpu.SemaphoreType.DMA] * _NBUF,
            [pltpu.SemaphoreType.DMA] * _NBUF,
        ],
    )
    def k(idx_hbm, table_hbm, out_hbm, idx_v, rows_v, isems, gsems, ssems):
        wid = lax.axis_index("s") * nc + lax.axis_index("c")
        base = wid * (chunk * n_chunks)

        for b in range(_NBUF):
            pltpu.async_copy(
                idx_hbm.at[pl.ds(base + b * chunk, chunk)], idx_v.at[b], isems[b]
            )

        def body(g, carry):
            for b in range(_NBUF):
                i = g * _NBUF + b

                @pl.when(g > 0)
                def _wait_store(b=b, i=i):
                    pltpu.make_async_copy(
                        rows_v.at[b],
                        out_hbm.at[pl.ds(base + (i - _NBUF) * chunk, chunk)],
                        ssems[b],
                    ).wait()

                pltpu.make_async_copy(
                    idx_hbm.at[pl.ds(base + i * chunk, chunk)],
                    idx_v.at[b],
                    isems[b],
                ).wait()
                pltpu.async_copy(table_hbm.at[idx_v.at[b]], rows_v.at[b], gsems[b])

            for b in range(_NBUF):
                i = g * _NBUF + b
                pltpu.make_async_copy(
                    table_hbm.at[idx_v.at[b]], rows_v.at[b], gsems[b]
                ).wait()
                pltpu.async_copy(
                    rows_v.at[b], out_hbm.at[pl.ds(base + i * chunk, chunk)], ssems[b]
                )

                @pl.when(g + 1 < n_super)
                def _prefetch_idx(b=b, i=i):
                    pltpu.async_copy(
                        idx_hbm.at[pl.ds(base + (i + _NBUF) * chunk, chunk)],
                        idx_v.at[b],
                        isems[b],
                    )

            return carry

        lax.fori_loop(0, n_super, body, 0)

        for b in range(_NBUF):
            i = (n_super - 1) * _NBUF + b
            pltpu.make_async_copy(
                rows_v.at[b],
                out_hbm.at[pl.ds(base + i * chunk, chunk)],
                ssems[b],
            ).wait()

    return k


def kernel(x, table):
    batch, hist = x.shape
    vocab, d_model = table.shape
    n_rows = batch * hist

    info = plsc.get_sparse_core_info()
    nw = info.num_cores * info.num_subcores
    rows_per_w = n_rows // nw
    chunk = 1600
    n_chunks = rows_per_w // chunk

    idx = x.reshape(n_rows).astype(jnp.int32)
    k = _gather_kernel(n_rows, d_model, chunk, n_chunks, info.num_cores)
    out = k(idx, table)
    return out.reshape(batch, hist, d_model)
